# baseline (speedup 1.0000x reference)
import functools

import jax
import jax.numpy as jnp
import numpy as np
from jax.experimental import pallas as pl
from jax.experimental.pallas import tpu as pltpu

VOCAB = 1024
EMBED = 256
LOG_PARAM_Q = -2.995732273553991
INV_TEMPERATURE = 2.0

BLOCK_ROWS = 512

_ROTATIONS = ((13, 15, 26, 6), (17, 29, 16, 24))


def _rotl(x, d):
    return jax.lax.shift_left(x, np.uint32(d)) | jax.lax.shift_right_logical(
        x, np.uint32(32 - d))


def _threefry_bits(x1):
    ks0 = np.uint32(0)
    ks1 = np.uint32(42)
    ks2 = np.uint32(0x1BD11BDA) ^ ks0 ^ ks1
    ks = (ks0, ks1, ks2)
    x0 = x1
    x1 = _rotl(x1, _ROTATIONS[0][0]) ^ x0
    for r in _ROTATIONS[0][1:]:
        x0 = x0 + x1
        x1 = _rotl(x1, r)
        x1 = x1 ^ x0
    x0 = x0 + ks[1]
    x1 = x1 + np.uint32(ks[2] + np.uint32(1))
    for i in range(1, 5):
        for r in _ROTATIONS[i % 2]:
            x0 = x0 + x1
            x1 = _rotl(x1, r)
            x1 = x1 ^ x0
        c0 = ks[(i + 1) % 3]
        if c0:
            x0 = x0 + c0
        x1 = x1 + np.uint32(ks[(i + 2) % 3] + np.uint32(i + 1))
    return x0 ^ x1


def _uniform_from_bits(bits):
    f = jax.lax.bitcast_convert_type(
        jax.lax.shift_right_logical(bits, np.uint32(9)) | np.uint32(0x3F800000),
        jnp.float32)
    return f - 1.0


def _body(lin_ref, z_ref, emb_ref, tok_ref, zq_ref, en_ref, *, block_rows,
          kappa):
    i = pl.program_id(0)

    @pl.when(i == 0)
    def _():
        emb = emb_ref[...]
        en_ref[...] = emb / jnp.maximum(
            jnp.sqrt(jnp.sum(emb * emb, axis=1, keepdims=True)), 1e-12)

    z = z_ref[...]
    zn = z / jnp.maximum(
        jnp.sqrt(jnp.sum(z * z, axis=1, keepdims=True)), 1e-12)
    zs = zn * np.float32(INV_TEMPERATURE * kappa / np.log(2.0))
    en = en_ref[...]
    en16 = en.astype(jnp.bfloat16)

    v = jax.lax.dot_general(
        zs.astype(jnp.bfloat16), en16, (((1,), (1,)), ((), ())),
        preferred_element_type=jnp.float32,
        precision=jax.lax.Precision.DEFAULT)

    base = (jnp.uint32(i) * np.uint32(block_rows * VOCAB) + np.uint32(42))
    u = _uniform_from_bits(_threefry_bits(lin_ref[...] + base))
    t = -jnp.log2(u)
    e = jax.lax.exp2(v - 2.0 * jnp.log2(t))
    tok = e * (1.0 / jnp.sum(e, axis=-1, keepdims=True))
    tok_ref[...] = tok

    zq_ref[...] = jax.lax.dot_general(
        tok.astype(jnp.bfloat16), en16, (((1,), (0,)), ((), ())),
        preferred_element_type=jnp.float32,
        precision=jax.lax.Precision.DEFAULT)


@jax.jit
def kernel(z, emb_weight):
    n = z.shape[0]
    block = BLOCK_ROWS
    grid = n // block
    kappa = float(np.exp(np.float32(LOG_PARAM_Q)).astype(np.float32)) + 1.0
    body = functools.partial(_body, block_rows=block, kappa=np.float32(kappa))
    lin = jnp.arange(block * VOCAB, dtype=jnp.uint32).reshape(block, VOCAB)
    tokens, z_q = pl.pallas_call(
        body,
        grid=(grid,),
        in_specs=[
            pl.BlockSpec((block, VOCAB), lambda i: (0, 0)),
            pl.BlockSpec((block, EMBED), lambda i: (i, 0)),
            pl.BlockSpec((VOCAB, EMBED), lambda i: (0, 0)),
        ],
        out_specs=[
            pl.BlockSpec((block, VOCAB), lambda i: (i, 0)),
            pl.BlockSpec((block, EMBED), lambda i: (i, 0)),
        ],
        out_shape=[
            jax.ShapeDtypeStruct((n, VOCAB), jnp.float32),
            jax.ShapeDtypeStruct((n, EMBED), jnp.float32),
        ],
        scratch_shapes=[pltpu.VMEM((VOCAB, EMBED), jnp.float32)],
        compiler_params=pltpu.CompilerParams(
            vmem_limit_bytes=120 * 1024 * 1024),
    )(lin, z, emb_weight)
    return tokens, z_q


# speedup vs baseline: 1.0133x; 1.0133x over previous
import functools

import jax
import jax.numpy as jnp
import numpy as np
from jax.experimental import pallas as pl
from jax.experimental.pallas import tpu as pltpu

VOCAB = 1024
EMBED = 256
LOG_PARAM_Q = -2.995732273553991
INV_TEMPERATURE = 2.0

BLOCK_ROWS = 1024

_ROTATIONS = ((13, 15, 26, 6), (17, 29, 16, 24))


def _rotl(x, d):
    return jax.lax.shift_left(x, np.uint32(d)) | jax.lax.shift_right_logical(
        x, np.uint32(32 - d))


def _threefry_bits(x1):
    ks0 = np.uint32(0)
    ks1 = np.uint32(42)
    ks2 = np.uint32(0x1BD11BDA) ^ ks0 ^ ks1
    ks = (ks0, ks1, ks2)
    x0 = x1
    x1 = _rotl(x1, _ROTATIONS[0][0]) ^ x0
    for r in _ROTATIONS[0][1:]:
        x0 = x0 + x1
        x1 = _rotl(x1, r)
        x1 = x1 ^ x0
    x0 = x0 + ks[1]
    x1 = x1 + np.uint32(ks[2] + np.uint32(1))
    for i in range(1, 5):
        for r in _ROTATIONS[i % 2]:
            x0 = x0 + x1
            x1 = _rotl(x1, r)
            x1 = x1 ^ x0
        c0 = ks[(i + 1) % 3]
        if c0:
            x0 = x0 + c0
        x1 = x1 + np.uint32(ks[(i + 2) % 3] + np.uint32(i + 1))
    return x0 ^ x1


def _uniform_from_bits(bits):
    f = jax.lax.bitcast_convert_type(
        jax.lax.shift_right_logical(bits, np.uint32(9)) | np.uint32(0x3F800000),
        jnp.float32)
    return f - 1.0


def _body(lin_ref, z_ref, emb_ref, tok_ref, zq_ref, en_ref, *, block_rows,
          kappa):
    i = pl.program_id(0)

    @pl.when(i == 0)
    def _():
        emb = emb_ref[...]
        en_ref[...] = emb / jnp.maximum(
            jnp.sqrt(jnp.sum(emb * emb, axis=1, keepdims=True)), 1e-12)

    z = z_ref[...]
    zn = z / jnp.maximum(
        jnp.sqrt(jnp.sum(z * z, axis=1, keepdims=True)), 1e-12)
    zs = zn * np.float32(INV_TEMPERATURE * kappa / np.log(2.0))
    en = en_ref[...]
    v = jax.lax.dot_general(
        zs, en, (((1,), (1,)), ((), ())),
        preferred_element_type=jnp.float32,
        precision=jax.lax.Precision.DEFAULT)

    base = (jnp.uint32(i) * np.uint32(block_rows * VOCAB) + np.uint32(42))
    u = _uniform_from_bits(_threefry_bits(lin_ref[...] + base))
    t = -jnp.log2(u)
    e = jax.lax.exp2(v - 2.0 * jnp.log2(t))
    tok = e * (1.0 / jnp.sum(e, axis=-1, keepdims=True))
    tok_ref[...] = tok

    zq_ref[...] = jax.lax.dot_general(
        tok, en, (((1,), (0,)), ((), ())),
        preferred_element_type=jnp.float32,
        precision=jax.lax.Precision.DEFAULT)


@jax.jit
def kernel(z, emb_weight):
    n = z.shape[0]
    block = BLOCK_ROWS
    grid = n // block
    kappa = float(np.exp(np.float32(LOG_PARAM_Q)).astype(np.float32)) + 1.0
    body = functools.partial(_body, block_rows=block, kappa=np.float32(kappa))
    lin = jnp.arange(block * VOCAB, dtype=jnp.uint32).reshape(block, VOCAB)
    tokens, z_q = pl.pallas_call(
        body,
        grid=(grid,),
        in_specs=[
            pl.BlockSpec((block, VOCAB), lambda i: (0, 0)),
            pl.BlockSpec((block, EMBED), lambda i: (i, 0)),
            pl.BlockSpec((VOCAB, EMBED), lambda i: (0, 0)),
        ],
        out_specs=[
            pl.BlockSpec((block, VOCAB), lambda i: (i, 0)),
            pl.BlockSpec((block, EMBED), lambda i: (i, 0)),
        ],
        out_shape=[
            jax.ShapeDtypeStruct((n, VOCAB), jnp.float32),
            jax.ShapeDtypeStruct((n, EMBED), jnp.float32),
        ],
        scratch_shapes=[pltpu.VMEM((VOCAB, EMBED), jnp.float32)],
        compiler_params=pltpu.CompilerParams(
            vmem_limit_bytes=120 * 1024 * 1024),
    )(lin, z, emb_weight)
    return tokens, z_q
